# baseline (speedup 1.0000x reference)
import numpy as np
import jax
import jax.numpy as jnp
from jax import lax
from jax.experimental import pallas as pl
from jax.experimental.pallas import tpu as pltpu
from jax.experimental.pallas import tpu_sc as plsc

B, H, W = 8, 512, 512
HW = H * W
N = B * HW
SAME = 0.2

NW = 32
RS2 = N // NW
RS = RS2 // 2
CH = 8192
NCH = HW // CH
NV = CH // 16
G = 32


def _idx_body(flow_ref, idx_ref):
    b = pl.program_id(0)
    fx = flow_ref[0, 0]
    fy = flow_ref[0, 1]
    zero = (fx == 0.0) & (fy == 0.0)
    fx = jnp.where(zero, 1000.0, fx)
    fy = jnp.where(zero, 1000.0, fy)
    gy = lax.broadcasted_iota(jnp.int32, (H, W), 0).astype(jnp.float32)
    gx = lax.broadcasted_iota(jnp.int32, (H, W), 1).astype(jnp.float32)
    ty = jnp.round(jnp.clip(gy + fy, 0.0, H - 1.0)).astype(jnp.int32)
    tx = jnp.round(jnp.clip(gx + fx, 0.0, W - 1.0)).astype(jnp.int32)
    idx_ref[0] = b * HW + ty * W + tx


_tc_idx = pl.pallas_call(
    _idx_body,
    grid=(B,),
    in_specs=[pl.BlockSpec((1, 2, H, W), lambda b: (b, 0, 0, 0))],
    out_specs=pl.BlockSpec((1, H, W), lambda b: (b, 0, 0)),
    out_shape=jax.ShapeDtypeStruct((B, H, W), jnp.int32),
)


def _sc_body(idx_hbm, d_hbm, o_hbm, out_hbm, minb, outb, idxc, dc, oc):
    c = lax.axis_index("c")
    s = lax.axis_index("s")
    w = s * 2 + c
    lo = w * RS2
    img = w // 4
    src0 = img * HW

    INF = jnp.float32(jnp.inf)

    def initmin(i, x):
        minb[pl.ds(i * 16, 16)] = jnp.full((16,), INF, jnp.float32)
        return x

    lax.fori_loop(0, RS2 // 16, initmin, 0, unroll=4)

    def chunkB(ci, x):
        base = src0 + ci * CH
        pltpu.sync_copy(idx_hbm.at[pl.ds(base, CH)], idxc)
        pltpu.sync_copy(d_hbm.at[pl.ds(base, CH)], dc)

        def grp(gi, y):
            addrs, ms, news = [], [], []
            for j in range(G):
                o16 = (gi * G + j) * 16
                iv = idxc[pl.ds(o16, 16)]
                dv = dc[pl.ds(o16, 16)]
                off = iv - lo
                m = (off >= 0) & (off < RS2)
                addr = jnp.where(m, off, 0)
                d = jnp.where(m, dv, INF)
                cur = plsc.load_gather(minb, [addr])
                addrs.append(addr)
                ms.append(m)
                news.append(jnp.minimum(cur, d))
            for j in range(G):
                plsc.store_scatter(minb, [addrs[j]], news[j], mask=ms[j])
            losts = []
            for j in range(G):
                back = plsc.load_gather(minb, [addrs[j]])
                losts.append(ms[j] & (back > news[j]))

            def cond(ls):
                any_l = ls[0]
                for l in ls[1:]:
                    any_l = any_l | l
                return plsc.all_reduce_population_count(any_l)[0] > 0

            def body(ls):
                for j in range(G):
                    plsc.store_scatter(minb, [addrs[j]], news[j],
                                       mask=ls[j])
                nls = []
                for j in range(G):
                    back = plsc.load_gather(minb, [addrs[j]])
                    nls.append(ms[j] & (back > news[j]))
                return tuple(nls)

            lax.while_loop(cond, body, tuple(losts))
            return y

        return lax.fori_loop(0, NV // G, grp, x)

    lax.fori_loop(0, NCH, chunkB, 0)

    for r in range(2):
        lo_r = lo + r * RS

        def initout(i, x):
            outb[pl.ds(i * 16, 16)] = jnp.full((16,), -INF, jnp.float32)
            return x

        lax.fori_loop(0, RS // 16, initout, 0, unroll=4)

        def chunkC(ci, x):
            base = src0 + ci * CH
            pltpu.sync_copy(idx_hbm.at[pl.ds(base, CH)], idxc)
            pltpu.sync_copy(d_hbm.at[pl.ds(base, CH)], dc)
            pltpu.sync_copy(o_hbm.at[pl.ds(base, CH)], oc)

            def grp(gi, y):
                addrs, ms, news = [], [], []
                for j in range(G):
                    o16 = (gi * G + j) * 16
                    iv = idxc[pl.ds(o16, 16)]
                    dv = dc[pl.ds(o16, 16)]
                    ov = oc[pl.ds(o16, 16)]
                    offr = iv - lo_r
                    m = (offr >= 0) & (offr < RS)
                    offb = jnp.where(m, iv - lo, 0)
                    mv = plsc.load_gather(minb, [offb])
                    val = jnp.where(m & (dv <= mv + SAME), ov, -INF)
                    addr = jnp.where(m, offr, 0)
                    cur = plsc.load_gather(outb, [addr])
                    addrs.append(addr)
                    ms.append(m)
                    news.append(jnp.maximum(cur, val))
                for j in range(G):
                    plsc.store_scatter(outb, [addrs[j]], news[j],
                                       mask=ms[j])
                losts = []
                for j in range(G):
                    back = plsc.load_gather(outb, [addrs[j]])
                    losts.append(ms[j] & (back < news[j]))

                def cond(ls):
                    any_l = ls[0]
                    for l in ls[1:]:
                        any_l = any_l | l
                    return plsc.all_reduce_population_count(any_l)[0] > 0

                def body(ls):
                    for j in range(G):
                        plsc.store_scatter(outb, [addrs[j]], news[j],
                                           mask=ls[j])
                    nls = []
                    for j in range(G):
                        back = plsc.load_gather(outb, [addrs[j]])
                        nls.append(ms[j] & (back < news[j]))
                    return tuple(nls)

                lax.while_loop(cond, body, tuple(losts))
                return y

            return lax.fori_loop(0, NV // G, grp, x)

        lax.fori_loop(0, NCH, chunkC, 0)

        def fixup(i, x):
            v = outb[pl.ds(i * 16, 16)]
            outb[pl.ds(i * 16, 16)] = jnp.where(jnp.abs(v) == INF, 0.0, v)
            return x

        lax.fori_loop(0, RS // 16, fixup, 0, unroll=4)
        pltpu.sync_copy(outb, out_hbm.at[pl.ds(lo_r, RS)])


_sc_scatter = pl.kernel(
    _sc_body,
    out_type=jax.ShapeDtypeStruct((N,), jnp.float32),
    mesh=plsc.VectorSubcoreMesh(core_axis_name="c", subcore_axis_name="s"),
    compiler_params=pltpu.CompilerParams(needs_layout_passes=False),
    scratch_types=[
        pltpu.VMEM((RS2,), jnp.float32),
        pltpu.VMEM((RS,), jnp.float32),
        pltpu.VMEM((CH,), jnp.int32),
        pltpu.VMEM((CH,), jnp.float32),
        pltpu.VMEM((CH,), jnp.float32),
    ],
)


@jax.jit
def kernel(obj, flow, depth):
    idx = _tc_idx(flow).reshape(N)
    out = _sc_scatter(idx, depth.reshape(N), obj.reshape(N))
    return out.reshape(B, 1, H, W)


# speedup vs baseline: 3.1303x; 3.1303x over previous
import numpy as np
import jax
import jax.numpy as jnp
from jax import lax
from jax.experimental import pallas as pl
from jax.experimental.pallas import tpu as pltpu
from jax.experimental.pallas import tpu_sc as plsc

B, H, W = 8, 512, 512
HW = H * W
N = B * HW
SAME = 0.2

NW = 32
RS2 = N // NW
RS = RS2 // 2
CH = 4096
NCH = HW // CH
NV = CH // 16
G = 16


def _idx_body(flow_ref, idx_ref):
    b = pl.program_id(0)
    fx = flow_ref[0, 0]
    fy = flow_ref[0, 1]
    zero = (fx == 0.0) & (fy == 0.0)
    fx = jnp.where(zero, 1000.0, fx)
    fy = jnp.where(zero, 1000.0, fy)
    gy = lax.broadcasted_iota(jnp.int32, (H, W), 0).astype(jnp.float32)
    gx = lax.broadcasted_iota(jnp.int32, (H, W), 1).astype(jnp.float32)
    ty = jnp.round(jnp.clip(gy + fy, 0.0, H - 1.0)).astype(jnp.int32)
    tx = jnp.round(jnp.clip(gx + fx, 0.0, W - 1.0)).astype(jnp.int32)
    idx_ref[0] = b * HW + ty * W + tx


_tc_idx = pl.pallas_call(
    _idx_body,
    grid=(B,),
    in_specs=[pl.BlockSpec((1, 2, H, W), lambda b: (b, 0, 0, 0))],
    out_specs=pl.BlockSpec((1, H, W), lambda b: (b, 0, 0)),
    out_shape=jax.ShapeDtypeStruct((B, H, W), jnp.int32),
)


def _sc_body(idx_hbm, d_hbm, o_hbm, out_hbm, minb, outb,
             idxc0, idxc1, dc0, dc1, oc0, oc1, sem0, sem1):
    c = lax.axis_index("c")
    s = lax.axis_index("s")
    w = s * 2 + c
    lo = w * RS2
    img = w // 4
    src0 = img * HW

    INF = jnp.float32(jnp.inf)
    bufs = ((idxc0, dc0, oc0, sem0), (idxc1, dc1, oc1, sem1))

    def start_fill(bs, base, with_o):
        idxc, dc, oc, sem = bs
        pltpu.async_copy(idx_hbm.at[pl.ds(base, CH)], idxc, sem)
        pltpu.async_copy(d_hbm.at[pl.ds(base, CH)], dc, sem)
        if with_o:
            pltpu.async_copy(o_hbm.at[pl.ds(base, CH)], oc, sem)

    def wait_fill(bs, with_o):
        idxc, dc, oc, sem = bs
        pltpu.make_async_copy(idx_hbm.at[pl.ds(0, CH)], idxc, sem).wait()
        pltpu.make_async_copy(d_hbm.at[pl.ds(0, CH)], dc, sem).wait()
        if with_o:
            pltpu.make_async_copy(o_hbm.at[pl.ds(0, CH)], oc, sem).wait()

    def initmin(i, x):
        minb[pl.ds(i * 16, 16)] = jnp.full((16,), INF, jnp.float32)
        return x

    lax.fori_loop(0, RS2 // 16, initmin, 0, unroll=4)

    def procB(bs):
        idxc, dc, _, _ = bs

        def grp(gi, y):
            addrs, ms, news = [], [], []
            for j in range(G):
                o16 = (gi * G + j) * 16
                iv = idxc[pl.ds(o16, 16)]
                dv = dc[pl.ds(o16, 16)]
                off = iv - lo
                m = (off >= 0) & (off < RS2)
                addr = jnp.where(m, off, 0)
                d = jnp.where(m, dv, INF)
                cur = plsc.load_gather(minb, [addr])
                addrs.append(addr)
                ms.append(m)
                news.append(jnp.minimum(cur, d))
            for j in range(G):
                plsc.store_scatter(minb, [addrs[j]], news[j], mask=ms[j])
            losts = []
            for j in range(G):
                back = plsc.load_gather(minb, [addrs[j]])
                losts.append(ms[j] & (back > news[j]))

            def cond(ls):
                any_l = ls[0]
                for l in ls[1:]:
                    any_l = any_l | l
                return plsc.all_reduce_population_count(any_l)[0] > 0

            def body(ls):
                for j in range(G):
                    plsc.store_scatter(minb, [addrs[j]], news[j],
                                       mask=ls[j])
                nls = []
                for j in range(G):
                    back = plsc.load_gather(minb, [addrs[j]])
                    nls.append(ms[j] & (back > news[j]))
                return tuple(nls)

            lax.while_loop(cond, body, tuple(losts))
            return y

        lax.fori_loop(0, NV // G, grp, 0)

    start_fill(bufs[0], src0, False)

    def chunk2B(q, x):
        start_fill(bufs[1], src0 + (2 * q + 1) * CH, False)
        wait_fill(bufs[0], False)
        procB(bufs[0])
        nxt = jnp.minimum(2 * q + 2, NCH - 1)
        start_fill(bufs[0], src0 + nxt * CH, False)
        wait_fill(bufs[1], False)
        procB(bufs[1])
        return x

    lax.fori_loop(0, NCH // 2, chunk2B, 0)
    wait_fill(bufs[0], False)

    for r in range(2):
        lo_r = lo + r * RS

        def initout(i, x):
            outb[pl.ds(i * 16, 16)] = jnp.full((16,), -INF, jnp.float32)
            return x

        lax.fori_loop(0, RS // 16, initout, 0, unroll=4)

        def procC(bs):
            idxc, dc, oc, _ = bs

            def grp(gi, y):
                addrs, ms, news = [], [], []
                for j in range(G):
                    o16 = (gi * G + j) * 16
                    iv = idxc[pl.ds(o16, 16)]
                    dv = dc[pl.ds(o16, 16)]
                    ov = oc[pl.ds(o16, 16)]
                    offr = iv - lo_r
                    m = (offr >= 0) & (offr < RS)
                    offb = jnp.where(m, iv - lo, 0)
                    mv = plsc.load_gather(minb, [offb])
                    val = jnp.where(m & (dv <= mv + SAME), ov, -INF)
                    addr = jnp.where(m, offr, 0)
                    cur = plsc.load_gather(outb, [addr])
                    addrs.append(addr)
                    ms.append(m)
                    news.append(jnp.maximum(cur, val))
                for j in range(G):
                    plsc.store_scatter(outb, [addrs[j]], news[j],
                                       mask=ms[j])
                losts = []
                for j in range(G):
                    back = plsc.load_gather(outb, [addrs[j]])
                    losts.append(ms[j] & (back < news[j]))

                def cond(ls):
                    any_l = ls[0]
                    for l in ls[1:]:
                        any_l = any_l | l
                    return plsc.all_reduce_population_count(any_l)[0] > 0

                def body(ls):
                    for j in range(G):
                        plsc.store_scatter(outb, [addrs[j]], news[j],
                                           mask=ls[j])
                    nls = []
                    for j in range(G):
                        back = plsc.load_gather(outb, [addrs[j]])
                        nls.append(ms[j] & (back < news[j]))
                    return tuple(nls)

                lax.while_loop(cond, body, tuple(losts))
                return y

            lax.fori_loop(0, NV // G, grp, 0)

        start_fill(bufs[0], src0, True)

        def chunk2C(q, x):
            start_fill(bufs[1], src0 + (2 * q + 1) * CH, True)
            wait_fill(bufs[0], True)
            procC(bufs[0])
            nxt = jnp.minimum(2 * q + 2, NCH - 1)
            start_fill(bufs[0], src0 + nxt * CH, True)
            wait_fill(bufs[1], True)
            procC(bufs[1])
            return x

        lax.fori_loop(0, NCH // 2, chunk2C, 0)
        wait_fill(bufs[0], True)

        def fixup(i, x):
            v = outb[pl.ds(i * 16, 16)]
            outb[pl.ds(i * 16, 16)] = jnp.where(jnp.abs(v) == INF, 0.0, v)
            return x

        lax.fori_loop(0, RS // 16, fixup, 0, unroll=4)
        pltpu.sync_copy(outb, out_hbm.at[pl.ds(lo_r, RS)])


_sc_scatter = pl.kernel(
    _sc_body,
    out_type=jax.ShapeDtypeStruct((N,), jnp.float32),
    mesh=plsc.VectorSubcoreMesh(core_axis_name="c", subcore_axis_name="s"),
    compiler_params=pltpu.CompilerParams(needs_layout_passes=False),
    scratch_types=[
        pltpu.VMEM((RS2,), jnp.float32),
        pltpu.VMEM((RS,), jnp.float32),
        pltpu.VMEM((CH,), jnp.int32),
        pltpu.VMEM((CH,), jnp.int32),
        pltpu.VMEM((CH,), jnp.float32),
        pltpu.VMEM((CH,), jnp.float32),
        pltpu.VMEM((CH,), jnp.float32),
        pltpu.VMEM((CH,), jnp.float32),
        pltpu.SemaphoreType.DMA,
        pltpu.SemaphoreType.DMA,
    ],
)


@jax.jit
def kernel(obj, flow, depth):
    idx = _tc_idx(flow).reshape(N)
    out = _sc_scatter(idx, depth.reshape(N), obj.reshape(N))
    return out.reshape(B, 1, H, W)
